# Initial kernel scaffold; baseline (speedup 1.0000x reference)
import jax
import jax.numpy as jnp
from jax.experimental import pallas as pl


def kernel(ent_emb, rel_emb, W1, W2, conv_w, conv_b, fc_w, fc_b, ent_bias, edge_weight, h_ids, r_ids, edge_index):
    raise NotImplementedError("write your pallas kernel here")


# speedup vs baseline: 7.8502x; 7.8502x over previous
import functools

import jax
import jax.numpy as jnp
from jax import lax
from jax.experimental import pallas as pl
from jax.experimental.pallas import tpu as pltpu
from jax.experimental.pallas import tpu_sc as plsc

NV = 10000
H = 128
NE = 320000
BS = 1024
NCORE = 2
NSUB = 16
NW = NCORE * NSUB
EPT = NE // NW
K = 80
NCH = EPT // K
SW = 16
OUT_CH = 32
KER = 7
POS = 100
FLAT = OUT_CH * POS
NVP = 10240
NVA = 10240


@functools.cache
def _get_sc_agg():
    mesh = plsc.VectorSubcoreMesh(core_axis_name="c", subcore_axis_name="s")

    @functools.partial(
        pl.kernel,
        mesh=mesh,
        compiler_params=pltpu.CompilerParams(needs_layout_passes=False),
        out_type=[jax.ShapeDtypeStruct((NCORE, NVA, H), jnp.float32),
                  jax.ShapeDtypeStruct((NW, NVA), jnp.float32)],
        scratch_types=[
            pltpu.VMEM((K,), jnp.int32),
            pltpu.VMEM((K,), jnp.int32),
            pltpu.VMEM((K, H), jnp.float32),
            pltpu.VMEM((K, H), jnp.float32),
            pltpu.VMEM((NVA,), jnp.float32),
            pltpu.VMEM_SHARED((NVA, H), jnp.float32),
            pltpu.SemaphoreType.DMA,
            pltpu.SemaphoreType.DMA,
        ],
    )
    def _sc_agg(emb_hbm, src_hbm, dst_hbm,
                racc_hbm, sparts_hbm,
                src_c, dst_c, srows, drows, s_local,
                acc_sh, sem1, sem2):
        cid = lax.axis_index("c")
        sid = lax.axis_index("s")
        wid = cid * NSUB + sid

        nrs = NVA // NSUB
        r0 = pl.multiple_of(sid * nrs, 8)

        def _zrow(e, _):
            for c in range(H // 16):
                srows[e, pl.ds(c * 16, 16)] = jnp.zeros((16,), jnp.float32)
            return 0
        lax.fori_loop(0, K, _zrow, 0)

        def _zs(i, _):
            s_local[pl.ds(i * 16, 16)] = jnp.zeros((16,), jnp.float32)
            return 0
        lax.fori_loop(0, NVA // 16, _zs, 0)

        iota16z = lax.iota(jnp.int32, 16)

        def _set_idx(base):
            for g in range(K // 16):
                src_c[pl.ds(g * 16, 16)] = base + g * 16 + iota16z

        def _zcopy(i, _):
            _set_idx(r0 + i * K)
            pltpu.sync_copy(srows, acc_sh.at[src_c])
            return 0
        lax.fori_loop(0, nrs // K, _zcopy, 0)

        plsc.subcore_barrier()

        iota16 = lax.iota(jnp.int32, 16)
        lane0 = iota16 == 0
        ebase = wid * EPT

        def _chunk(j, _):
            pltpu.sync_copy(src_hbm.at[pl.ds(ebase + j * K, K)], src_c)
            pltpu.sync_copy(dst_hbm.at[pl.ds(ebase + j * K, K)], dst_c)
            cp1 = pltpu.async_copy(emb_hbm.at[src_c], srows, sem1)
            cp2 = pltpu.async_copy(emb_hbm.at[dst_c], drows, sem2)
            cp1.wait()
            cp2.wait()

            def _edges(g, _g):
                dvec = dst_c[pl.ds(g * 16, 16)]
                for u in range(16):
                    e = g * 16 + u
                    acc = jnp.zeros((16,), jnp.float32)
                    for c in range(H // 16):
                        acc = acc + (srows[e, pl.ds(c * 16, 16)]
                                     * drows[e, pl.ds(c * 16, 16)])
                    for s in (8, 4, 2, 1):
                        perm = jnp.bitwise_and(iota16 + s, 15)
                        acc = acc + acc.at[perm].get(
                            mode="promise_in_bounds")
                    ev = jnp.exp(acc)
                    for c in range(H // 16):
                        srows[e, pl.ds(c * 16, 16)] = (
                            srows[e, pl.ds(c * 16, 16)] * ev)
                    dsplat = dvec.at[jnp.full((16,), u, jnp.int32)].get(
                        mode="promise_in_bounds")
                    plsc.addupdate_scatter(s_local, [dsplat], ev,
                                           mask=lane0)
                return 0

            lax.fori_loop(0, K // 16, _edges, 0)
            pltpu.sync_copy(srows, acc_sh.at[dst_c], add=True)
            return 0

        lax.fori_loop(0, NCH, _chunk, 0)
        plsc.subcore_barrier()

        def _fcopy(i, _):
            rr = pl.multiple_of(r0 + i * K, 8)
            _set_idx(rr)
            pltpu.sync_copy(acc_sh.at[src_c], srows)
            pltpu.sync_copy(srows, racc_hbm.at[cid, pl.ds(rr, K)])
            return 0
        lax.fori_loop(0, nrs // K, _fcopy, 0)
        pltpu.sync_copy(s_local, sparts_hbm.at[wid])

    return _sc_agg


@functools.cache
def _get_sc_gather2():
    mesh = plsc.VectorSubcoreMesh(core_axis_name="c", subcore_axis_name="s")

    @functools.partial(
        pl.kernel,
        mesh=mesh,
        compiler_params=pltpu.CompilerParams(needs_layout_passes=False),
        out_type=[jax.ShapeDtypeStruct((BS, H), jnp.float32),
                  jax.ShapeDtypeStruct((BS, H), jnp.float32)],
        scratch_types=[
            pltpu.VMEM((BS // NW,), jnp.int32),
            pltpu.VMEM((BS // NW, H), jnp.float32),
            pltpu.SemaphoreType.DMA,
        ],
    )
    def _sc_gather2(emb_hbm, rel_hbm, hids_hbm, rids_hbm, head_out, rel_out,
                    idx_v, rows_v, sem):
        wid = lax.axis_index("c") * NSUB + lax.axis_index("s")
        b = BS // NW
        base = wid * b
        pltpu.sync_copy(hids_hbm.at[pl.ds(base, b)], idx_v)
        pltpu.async_copy(emb_hbm.at[idx_v], rows_v, sem).wait()
        pltpu.sync_copy(rows_v, head_out.at[pl.ds(base, b)])
        pltpu.sync_copy(rids_hbm.at[pl.ds(base, b)], idx_v)
        pltpu.async_copy(rel_hbm.at[idx_v], rows_v, sem).wait()
        pltpu.sync_copy(rows_v, rel_out.at[pl.ds(base, b)])

    return _sc_gather2


def _finalize(racc0, racc1, sparts, emb_old, W):
    RB = 1000

    def body(r0_ref, r1_ref, sp_ref, emb_ref, w_ref, out_ref):
        rows = r0_ref[...] + r1_ref[...]
        s = jnp.sum(sp_ref[...], axis=1)
        neigh = jnp.where(s[:, None] > 0, rows / s[:, None], 0.0)
        h = lax.dot_general(neigh, w_ref[...], (((1,), (0,)), ((), ())),
                            preferred_element_type=jnp.float32,
                            precision=lax.Precision.HIGHEST)
        out_ref[...] = emb_ref[...] + jnp.tanh(h)

    return pl.pallas_call(
        body,
        grid=(NV // RB,),
        in_specs=[
            pl.BlockSpec((RB, H), lambda i: (i, 0)),
            pl.BlockSpec((RB, H), lambda i: (i, 0)),
            pl.BlockSpec((RB, NW), lambda i: (i, 0)),
            pl.BlockSpec((RB, H), lambda i: (i, 0)),
            pl.BlockSpec((H, H), lambda i: (0, 0)),
        ],
        out_specs=pl.BlockSpec((RB, H), lambda i: (i, 0)),
        out_shape=jax.ShapeDtypeStruct((NV, H), jnp.float32),
    )(racc0, racc1, sparts, emb_old, W)


def _convfc(head, rel, wbig, b1, fcwp, fcb):
    RB = 256

    def body(h_ref, r_ref, wb_ref, b1_ref, fw_ref, fb_ref, out_ref):
        x = jnp.concatenate([h_ref[...], r_ref[...]], axis=1)
        x1 = lax.dot_general(x, wb_ref[...], (((1,), (0,)), ((), ())),
                             preferred_element_type=jnp.float32,
                             precision=lax.Precision.HIGHEST)
        x1 = jnp.maximum(x1 + b1_ref[...], 0.0)
        x2 = lax.dot_general(x1, fw_ref[...], (((1,), (0,)), ((), ())),
                             preferred_element_type=jnp.float32,
                             precision=lax.Precision.HIGHEST)
        out_ref[...] = jnp.maximum(x2 + fb_ref[...], 0.0)

    return pl.pallas_call(
        body,
        grid=(BS // RB,),
        in_specs=[
            pl.BlockSpec((RB, H), lambda i: (i, 0)),
            pl.BlockSpec((RB, H), lambda i: (i, 0)),
            pl.BlockSpec((2 * H, FLAT), lambda i: (0, 0)),
            pl.BlockSpec((1, FLAT), lambda i: (0, 0)),
            pl.BlockSpec((FLAT, H), lambda i: (0, 0)),
            pl.BlockSpec((1, H), lambda i: (0, 0)),
        ],
        out_specs=pl.BlockSpec((RB, H), lambda i: (i, 0)),
        out_shape=jax.ShapeDtypeStruct((BS, H), jnp.float32),
    )(head, rel, wbig, b1, fcwp, fcb)


def _score(x2, emb_pad, bias_pad):
    CB = 1024

    def body(x_ref, e_ref, b_ref, out_ref):
        h = lax.dot_general(x_ref[...], e_ref[...], (((1,), (1,)), ((), ())),
                            preferred_element_type=jnp.float32,
                            precision=lax.Precision.HIGHEST)
        out_ref[...] = jax.nn.sigmoid(h + b_ref[...])

    return pl.pallas_call(
        body,
        grid=(NVP // CB,),
        in_specs=[
            pl.BlockSpec((BS, H), lambda i: (0, 0)),
            pl.BlockSpec((CB, H), lambda i: (i, 0)),
            pl.BlockSpec((1, CB), lambda i: (0, i)),
        ],
        out_specs=pl.BlockSpec((BS, CB), lambda i: (0, i)),
        out_shape=jax.ShapeDtypeStruct((BS, NVP), jnp.float32),
    )(x2, emb_pad, bias_pad)


def _build_wbig(conv_w):
    A = conv_w[:, 0]
    r = jnp.arange(16)[:, None]
    i = jnp.arange(10)[None, :]
    ki = r - i
    vi = (ki >= 0) & (ki < KER)
    kic = jnp.clip(ki, 0, KER - 1)
    B = A[:, kic, :]
    C = B[:, :, :, kic.reshape(16, 10)]
    mask = vi[None, :, :, None, None] & vi[None, None, None, :, :]
    W6 = jnp.where(mask, C, 0.0)
    return W6.transpose(1, 3, 2, 4, 0).reshape(2 * H, FLAT)


def kernel(ent_emb, rel_emb, W1, W2, conv_w, conv_b, fc_w, fc_b, ent_bias,
           edge_weight, h_ids, r_ids, edge_index):
    src = edge_index[0].astype(jnp.int32)
    dst = edge_index[1].astype(jnp.int32)

    emb = ent_emb
    for W in (W1, W2):
        racc, sparts = _get_sc_agg()(emb, src, dst)
        emb = _finalize(racc[0, :NV], racc[1, :NV],
                        sparts[:, :NV].T, emb, W)

    head, rel = _get_sc_gather2()(emb, rel_emb,
                                  h_ids.astype(jnp.int32),
                                  r_ids.astype(jnp.int32))

    wbig = _build_wbig(conv_w)
    b1 = jnp.tile(conv_b, POS).reshape(1, FLAT)
    fcwp = fc_w.reshape(OUT_CH, POS, H).transpose(1, 0, 2).reshape(FLAT, H)
    x2 = _convfc(head, rel, wbig, b1, fcwp, fc_b.reshape(1, H))

    emb_pad = jnp.concatenate([emb, jnp.zeros((NVP - NV, H), jnp.float32)], axis=0)
    bias_pad = jnp.concatenate(
        [ent_bias, jnp.zeros((NVP - NV,), jnp.float32)]).reshape(1, NVP)
    score = _score(x2, emb_pad, bias_pad)[:, :NV]
    return (score, emb, rel_emb)


# speedup vs baseline: 7.8570x; 1.0009x over previous
import functools

import jax
import jax.numpy as jnp
from jax import lax
from jax.experimental import pallas as pl
from jax.experimental.pallas import tpu as pltpu
from jax.experimental.pallas import tpu_sc as plsc

NV = 10000
H = 128
NE = 320000
BS = 1024
NCORE = 2
NSUB = 16
NW = NCORE * NSUB
EPT = NE // NW
K = 80
NCH = EPT // K
OUT_CH = 32
KER = 7
POS = 100
FLAT = OUT_CH * POS
NVP = 10240
NVA = 10240


@functools.cache
def _get_sc_agg():
    mesh = plsc.VectorSubcoreMesh(core_axis_name="c", subcore_axis_name="s")

    @functools.partial(
        pl.kernel,
        mesh=mesh,
        compiler_params=pltpu.CompilerParams(needs_layout_passes=False),
        out_type=[jax.ShapeDtypeStruct((NCORE, NVA, H), jnp.float32),
                  jax.ShapeDtypeStruct((NW, NVA), jnp.float32)],
        scratch_types=[
            pltpu.VMEM((K,), jnp.int32),
            pltpu.VMEM((K,), jnp.int32),
            pltpu.VMEM((K, H), jnp.float32),
            pltpu.VMEM((K, H), jnp.float32),
            pltpu.VMEM((NVA,), jnp.float32),
            pltpu.VMEM_SHARED((NVA, H), jnp.float32),
            pltpu.SemaphoreType.DMA,
            pltpu.SemaphoreType.DMA,
        ],
    )
    def _sc_agg(emb_hbm, src_hbm, dst_hbm,
                racc_hbm, sparts_hbm,
                src_c, dst_c, srows, drows, s_local,
                acc_sh, sem1, sem2):
        cid = lax.axis_index("c")
        sid = lax.axis_index("s")
        wid = cid * NSUB + sid

        nrs = NVA // NSUB
        r0 = pl.multiple_of(sid * nrs, 8)

        def _zrow(e, _):
            for c in range(H // 16):
                srows[e, pl.ds(c * 16, 16)] = jnp.zeros((16,), jnp.float32)
            return 0
        lax.fori_loop(0, K, _zrow, 0)

        def _zs(i, _):
            s_local[pl.ds(i * 16, 16)] = jnp.zeros((16,), jnp.float32)
            return 0
        lax.fori_loop(0, NVA // 16, _zs, 0)

        iota16z = lax.iota(jnp.int32, 16)

        def _set_idx(base):
            for g in range(K // 16):
                src_c[pl.ds(g * 16, 16)] = base + g * 16 + iota16z

        def _zcopy(i, _):
            _set_idx(r0 + i * K)
            pltpu.sync_copy(srows, acc_sh.at[src_c])
            return 0
        lax.fori_loop(0, nrs // K, _zcopy, 0)

        plsc.subcore_barrier()

        iota16 = lax.iota(jnp.int32, 16)
        lane0 = iota16 == 0
        ebase = wid * EPT

        def _chunk(j, _):
            pltpu.sync_copy(src_hbm.at[pl.ds(ebase + j * K, K)], src_c)
            pltpu.sync_copy(dst_hbm.at[pl.ds(ebase + j * K, K)], dst_c)
            cp1 = pltpu.async_copy(emb_hbm.at[src_c], srows, sem1)
            cp2 = pltpu.async_copy(emb_hbm.at[dst_c], drows, sem2)
            cp1.wait()
            cp2.wait()

            def _edges(g, _g):
                dvec = dst_c[pl.ds(g * 16, 16)]
                for u in range(16):
                    e = g * 16 + u
                    acc = jnp.zeros((16,), jnp.float32)
                    for c in range(H // 16):
                        acc = acc + (srows[e, pl.ds(c * 16, 16)]
                                     * drows[e, pl.ds(c * 16, 16)])
                    for s in (8, 4, 2, 1):
                        perm = jnp.bitwise_and(iota16 + s, 15)
                        acc = acc + acc.at[perm].get(
                            mode="promise_in_bounds")
                    ev = jnp.exp(acc)
                    for c in range(H // 16):
                        srows[e, pl.ds(c * 16, 16)] = (
                            srows[e, pl.ds(c * 16, 16)] * ev)
                    dsplat = dvec.at[jnp.full((16,), u, jnp.int32)].get(
                        mode="promise_in_bounds")
                    plsc.addupdate_scatter(s_local, [dsplat], ev,
                                           mask=lane0)
                return 0

            lax.fori_loop(0, K // 16, _edges, 0)
            pltpu.sync_copy(srows, acc_sh.at[dst_c], add=True)
            return 0

        lax.fori_loop(0, NCH, _chunk, 0)
        plsc.subcore_barrier()

        def _fcopy(i, _):
            rr = pl.multiple_of(r0 + i * K, 8)
            _set_idx(rr)
            pltpu.sync_copy(acc_sh.at[src_c], srows)
            pltpu.sync_copy(srows, racc_hbm.at[cid, pl.ds(rr, K)])
            return 0
        lax.fori_loop(0, nrs // K, _fcopy, 0)
        pltpu.sync_copy(s_local, sparts_hbm.at[wid])

    return _sc_agg


@functools.cache
def _get_sc_gather2():
    mesh = plsc.VectorSubcoreMesh(core_axis_name="c", subcore_axis_name="s")

    @functools.partial(
        pl.kernel,
        mesh=mesh,
        compiler_params=pltpu.CompilerParams(needs_layout_passes=False),
        out_type=[jax.ShapeDtypeStruct((BS, H), jnp.float32),
                  jax.ShapeDtypeStruct((BS, H), jnp.float32)],
        scratch_types=[
            pltpu.VMEM((BS // NW,), jnp.int32),
            pltpu.VMEM((BS // NW, H), jnp.float32),
            pltpu.SemaphoreType.DMA,
        ],
    )
    def _sc_gather2(emb_hbm, rel_hbm, hids_hbm, rids_hbm, head_out, rel_out,
                    idx_v, rows_v, sem):
        wid = lax.axis_index("c") * NSUB + lax.axis_index("s")
        b = BS // NW
        base = wid * b
        pltpu.sync_copy(hids_hbm.at[pl.ds(base, b)], idx_v)
        pltpu.async_copy(emb_hbm.at[idx_v], rows_v, sem).wait()
        pltpu.sync_copy(rows_v, head_out.at[pl.ds(base, b)])
        pltpu.sync_copy(rids_hbm.at[pl.ds(base, b)], idx_v)
        pltpu.async_copy(rel_hbm.at[idx_v], rows_v, sem).wait()
        pltpu.sync_copy(rows_v, rel_out.at[pl.ds(base, b)])

    return _sc_gather2


def _finalize(racc0, racc1, sparts, emb_old, W):
    RB = 1000

    def body(r0_ref, r1_ref, sp_ref, emb_ref, w_ref, out_ref):
        rows = r0_ref[...] + r1_ref[...]
        s = jnp.sum(sp_ref[...], axis=1)
        neigh = jnp.where(s[:, None] > 0, rows / s[:, None], 0.0)
        h = lax.dot_general(neigh, w_ref[...], (((1,), (0,)), ((), ())),
                            preferred_element_type=jnp.float32,
                            precision=lax.Precision.HIGHEST)
        out_ref[...] = emb_ref[...] + jnp.tanh(h)

    return pl.pallas_call(
        body,
        grid=(NV // RB,),
        in_specs=[
            pl.BlockSpec((RB, H), lambda i: (i, 0)),
            pl.BlockSpec((RB, H), lambda i: (i, 0)),
            pl.BlockSpec((RB, NW), lambda i: (i, 0)),
            pl.BlockSpec((RB, H), lambda i: (i, 0)),
            pl.BlockSpec((H, H), lambda i: (0, 0)),
        ],
        out_specs=pl.BlockSpec((RB, H), lambda i: (i, 0)),
        out_shape=jax.ShapeDtypeStruct((NV, H), jnp.float32),
    )(racc0, racc1, sparts, emb_old, W)


def _convfc(head, rel, wbig, b1, fcwp, fcb):
    RB = 256

    def body(h_ref, r_ref, wb_ref, b1_ref, fw_ref, fb_ref, out_ref):
        x = jnp.concatenate([h_ref[...], r_ref[...]], axis=1)
        x1 = lax.dot_general(x, wb_ref[...], (((1,), (0,)), ((), ())),
                             preferred_element_type=jnp.float32,
                             precision=lax.Precision.HIGHEST)
        x1 = jnp.maximum(x1 + b1_ref[...], 0.0)
        x2 = lax.dot_general(x1, fw_ref[...], (((1,), (0,)), ((), ())),
                             preferred_element_type=jnp.float32,
                             precision=lax.Precision.HIGHEST)
        out_ref[...] = jnp.maximum(x2 + fb_ref[...], 0.0)

    return pl.pallas_call(
        body,
        grid=(BS // RB,),
        in_specs=[
            pl.BlockSpec((RB, H), lambda i: (i, 0)),
            pl.BlockSpec((RB, H), lambda i: (i, 0)),
            pl.BlockSpec((2 * H, FLAT), lambda i: (0, 0)),
            pl.BlockSpec((1, FLAT), lambda i: (0, 0)),
            pl.BlockSpec((FLAT, H), lambda i: (0, 0)),
            pl.BlockSpec((1, H), lambda i: (0, 0)),
        ],
        out_specs=pl.BlockSpec((RB, H), lambda i: (i, 0)),
        out_shape=jax.ShapeDtypeStruct((BS, H), jnp.float32),
    )(head, rel, wbig, b1, fcwp, fcb)


def _score(x2, emb_pad, bias_pad):
    CB = 1024

    def body(x_ref, e_ref, b_ref, out_ref):
        h = lax.dot_general(x_ref[...], e_ref[...], (((1,), (1,)), ((), ())),
                            preferred_element_type=jnp.float32,
                            precision=lax.Precision.HIGHEST)
        out_ref[...] = jax.nn.sigmoid(h + b_ref[...])

    return pl.pallas_call(
        body,
        grid=(NVP // CB,),
        in_specs=[
            pl.BlockSpec((BS, H), lambda i: (0, 0)),
            pl.BlockSpec((CB, H), lambda i: (i, 0)),
            pl.BlockSpec((1, CB), lambda i: (0, i)),
        ],
        out_specs=pl.BlockSpec((BS, CB), lambda i: (0, i)),
        out_shape=jax.ShapeDtypeStruct((BS, NVP), jnp.float32),
    )(x2, emb_pad, bias_pad)


def _build_wbig(conv_w):
    A = conv_w[:, 0]
    r = jnp.arange(16)[:, None]
    i = jnp.arange(10)[None, :]
    ki = r - i
    vi = (ki >= 0) & (ki < KER)
    kic = jnp.clip(ki, 0, KER - 1)
    B = A[:, kic, :]
    C = B[:, :, :, kic.reshape(16, 10)]
    mask = vi[None, :, :, None, None] & vi[None, None, None, :, :]
    W6 = jnp.where(mask, C, 0.0)
    return W6.transpose(1, 3, 2, 4, 0).reshape(2 * H, FLAT)


def kernel(ent_emb, rel_emb, W1, W2, conv_w, conv_b, fc_w, fc_b, ent_bias,
           edge_weight, h_ids, r_ids, edge_index):
    src = edge_index[0].astype(jnp.int32)
    dst = edge_index[1].astype(jnp.int32)

    emb = ent_emb
    for W in (W1, W2):
        racc, sparts = _get_sc_agg()(emb, src, dst)
        emb = _finalize(racc[0, :NV], racc[1, :NV],
                        sparts[:, :NV].T, emb, W)

    head, rel = _get_sc_gather2()(emb, rel_emb,
                                  h_ids.astype(jnp.int32),
                                  r_ids.astype(jnp.int32))

    wbig = _build_wbig(conv_w)
    b1 = jnp.tile(conv_b, POS).reshape(1, FLAT)
    fcwp = fc_w.reshape(OUT_CH, POS, H).transpose(1, 0, 2).reshape(FLAT, H)
    x2 = _convfc(head, rel, wbig, b1, fcwp, fc_b.reshape(1, H))

    emb_pad = jnp.concatenate([emb, jnp.zeros((NVP - NV, H), jnp.float32)], axis=0)
    bias_pad = jnp.concatenate(
        [ent_bias, jnp.zeros((NVP - NV,), jnp.float32)]).reshape(1, NVP)
    score = _score(x2, emb_pad, bias_pad)[:, :NV]
    return (score, emb, rel_emb)
